# Initial kernel scaffold; baseline (speedup 1.0000x reference)
import jax
import jax.numpy as jnp
from jax.experimental import pallas as pl


def kernel(x, edge_index, edge_attr, node_table, edge_table, node_gamma, node_beta, edge_gamma, edge_beta):
    raise NotImplementedError("write your pallas kernel here")


# speedup vs baseline: 8.2353x; 8.2353x over previous
import jax
import jax.numpy as jnp
from jax import lax
from jax.experimental import pallas as pl
from jax.experimental.pallas import tpu as pltpu
from jax.experimental.pallas import tpu_sc as plsc

N_NODES_K = 100000
N_EDGES_K = 3200000
DIM_K = 128
EDIM_K = 16
NV_K = 50000
EV_K = 5000
EPS_K = 1e-5

NC = 2
NS = 16
NW = NC * NS
L = 16


EV_PAD = 5120
NV_PAD = 50176
E_PER_HTILE = N_EDGES_K // NS
E_HCHUNK = 8000
NX_NOM = N_NODES_K // NS
NX_WIN = 6256
NCOPY_W = 3136
ECOPY_W = EV_PAD // NS

_sc_mesh = plsc.VectorSubcoreMesh(core_axis_name="c", subcore_axis_name="s")


def _hist_body(x_hbm, ea_hbm, cn_hbm, ce_hbm,
               ehist_v, exbuf_v, nxbuf_v, nvals_v, zbuf_v, acc_v, tmp_v,
               sh_e, sh_n):
    c = lax.axis_index("c")
    s = lax.axis_index("s")
    is_edge = c == 0

    @pl.when(is_edge)
    def _():
        def zero_body(j, carry):
            ehist_v[pl.ds(j * L, L)] = jnp.zeros((L,), jnp.float32)
            return carry
        lax.fori_loop(0, EV_PAD // L, zero_body, 0)

    @pl.when(jnp.logical_not(is_edge))
    def _():
        def zb(j, carry):
            zbuf_v[pl.ds(j * L, L)] = jnp.zeros((L,), jnp.float32)
            return carry
        lax.fori_loop(0, NCOPY_W // L, zb, 0)
        pltpu.sync_copy(zbuf_v, sh_n.at[pl.ds(s * NCOPY_W, NCOPY_W)])
        lo = s * NX_NOM
        start = pl.multiple_of(lo - lax.rem(lo, 8), 8)
        pltpu.sync_copy(x_hbm.at[pl.ds(start, NX_WIN)], nxbuf_v)
        hi = lo + NX_NOM
        lane = lax.iota(jnp.int32, L)
        def vb(j, carry):
            pos = start + j * L + lane
            m = (pos >= lo) & (pos < hi)
            nvals_v[pl.ds(j * L, L)] = jnp.where(m, 1.0, 0.0).astype(jnp.float32)
            return carry
        lax.fori_loop(0, NX_WIN // L, vb, 0)

    plsc.subcore_barrier()

    @pl.when(is_edge)
    def _():
        base = s * E_PER_HTILE
        ones = jnp.ones((L,), jnp.float32)
        def chunk(i, carry):
            pltpu.sync_copy(ea_hbm.at[pl.ds(base + i * E_HCHUNK, E_HCHUNK)],
                            exbuf_v)
            def inner(j, carry2):
                iv = exbuf_v[pl.ds(j * L, L)]
                plsc.addupdate_scatter(ehist_v, [iv], ones)
                return carry2
            lax.fori_loop(0, E_HCHUNK // L, inner, 0)
            return carry
        lax.fori_loop(0, E_PER_HTILE // E_HCHUNK, chunk, 0)
        pltpu.sync_copy(ehist_v,
                        sh_e.at[pl.ds(pl.multiple_of(s * EV_PAD, 8), EV_PAD)])

    @pl.when(jnp.logical_not(is_edge))
    def _():
        pltpu.sync_copy(nvals_v, sh_n.at[nxbuf_v], add=True)

    plsc.subcore_barrier()

    @pl.when(is_edge)
    def _():
        estart = pl.multiple_of(
            jnp.minimum(s * ECOPY_W, EV_K - ECOPY_W), 8)
        def zb(j, carry):
            acc_v[pl.ds(j * L, L)] = jnp.zeros((L,), jnp.float32)
            return carry
        lax.fori_loop(0, ECOPY_W // L, zb, 0)
        def red(k, carry):
            pltpu.sync_copy(
                sh_e.at[pl.ds(pl.multiple_of(k * EV_PAD + estart, 8),
                              ECOPY_W)], tmp_v)
            def addv(j, carry2):
                acc_v[pl.ds(j * L, L)] = (acc_v[pl.ds(j * L, L)]
                                          + tmp_v[pl.ds(j * L, L)])
                return carry2
            lax.fori_loop(0, ECOPY_W // L, addv, 0)
            return carry
        lax.fori_loop(0, NS, red, 0)
        pltpu.sync_copy(acc_v, ce_hbm.at[pl.ds(estart, ECOPY_W)])

    @pl.when(jnp.logical_not(is_edge))
    def _():
        nlo = s * (NV_K // NS)
        nstart = pl.multiple_of(
            jnp.minimum(nlo - lax.rem(nlo, 8), NV_K - NCOPY_W), 8)
        pltpu.sync_copy(sh_n.at[pl.ds(nstart, NCOPY_W)], zbuf_v)
        pltpu.sync_copy(zbuf_v, cn_hbm.at[pl.ds(nstart, NCOPY_W)])


_hist = pl.kernel(
    _hist_body,
    out_type=(jax.ShapeDtypeStruct((NV_K,), jnp.float32),
              jax.ShapeDtypeStruct((EV_K,), jnp.float32)),
    mesh=_sc_mesh,
    compiler_params=pltpu.CompilerParams(needs_layout_passes=False),
    scratch_types=[
        pltpu.VMEM((EV_PAD,), jnp.float32),
        pltpu.VMEM((E_HCHUNK,), jnp.int32),
        pltpu.VMEM((NX_WIN,), jnp.int32),
        pltpu.VMEM((NX_WIN,), jnp.float32),
        pltpu.VMEM((NCOPY_W,), jnp.float32),
        pltpu.VMEM((ECOPY_W,), jnp.float32),
        pltpu.VMEM((ECOPY_W,), jnp.float32),
        pltpu.VMEM_SHARED((NS * EV_PAD,), jnp.float32),
        pltpu.VMEM_SHARED((NV_PAD,), jnp.float32),
    ],
)


VBLK = 400
NBLK = NV_K // VBLK


def _nstats_body(c_ref, t_ref, o_ref, acc):
    j = pl.program_id(0)

    @pl.when(j == 0)
    def _():
        acc[...] = jnp.zeros_like(acc)

    cvec = c_ref[0]
    T = t_ref[...]
    s1 = jnp.dot(cvec, T, preferred_element_type=jnp.float32)
    s2 = jnp.dot(cvec, T * T, preferred_element_type=jnp.float32)
    acc[0:1, :] = acc[0:1, :] + s1
    acc[1:2, :] = acc[1:2, :] + s2
    o_ref[...] = acc[...]


_nstats = pl.pallas_call(
    _nstats_body,
    grid=(NBLK,),
    in_specs=[pl.BlockSpec((1, 1, VBLK), lambda j: (j, 0, 0)),
              pl.BlockSpec((VBLK, DIM_K), lambda j: (j, 0))],
    out_specs=pl.BlockSpec((2, DIM_K), lambda j: (0, 0)),
    out_shape=jax.ShapeDtypeStruct((2, DIM_K), jnp.float32),
    scratch_shapes=[pltpu.VMEM((2, DIM_K), jnp.float32)],
)


def _nnorm_body(ss_ref, g_ref, b_ref, t_ref, o_ref):
    s1 = ss_ref[0:1, :]
    s2 = ss_ref[1:2, :]
    mean = s1 / N_NODES_K
    var = s2 / N_NODES_K - mean * mean
    a = g_ref[...] * lax.rsqrt(var + EPS_K)
    b = b_ref[...] - mean * a
    o_ref[...] = t_ref[...] * a + b


_nnorm = pl.pallas_call(
    _nnorm_body,
    grid=(NBLK,),
    in_specs=[pl.BlockSpec((2, DIM_K), lambda j: (0, 0)),
              pl.BlockSpec((1, DIM_K), lambda j: (0, 0)),
              pl.BlockSpec((1, DIM_K), lambda j: (0, 0)),
              pl.BlockSpec((VBLK, DIM_K), lambda j: (j, 0))],
    out_specs=pl.BlockSpec((VBLK, DIM_K), lambda j: (j, 0)),
    out_shape=jax.ShapeDtypeStruct((NV_K, DIM_K), jnp.float32),
)


def _enorm_body(c_ref, g_ref, b_ref, t_ref, o_ref):
    cvec = c_ref[...]
    T = t_ref[...]
    s1 = jnp.dot(cvec, T, preferred_element_type=jnp.float32)
    s2 = jnp.dot(cvec, T * T, preferred_element_type=jnp.float32)
    mean = s1 / N_EDGES_K
    var = s2 / N_EDGES_K - mean * mean
    a = g_ref[...] * lax.rsqrt(var + EPS_K)
    b = b_ref[...] - mean * a
    o_ref[...] = T * a + b


_enorm = pl.pallas_call(
    _enorm_body,
    out_shape=jax.ShapeDtypeStruct((EV_K, EDIM_K), jnp.float32),
)


NG_NOM = N_NODES_K // NW
NG_WIN = 3136
NG_CHUNK = 392
E_PER_GTILE = N_EDGES_K // NW
EG_CHUNK = 2000


def _gather_body(nt_hbm, x_hbm, et_hbm, ea_hbm, hn_hbm, he_hbm,
                 nx_v, nrows_v, ex_v, erows_v, sem):
    c = lax.axis_index("c")
    s = lax.axis_index("s")
    w = s * NC + c

    nlo = w * NG_NOM
    nstart = pl.multiple_of(
        jnp.minimum(nlo - lax.rem(nlo, 8), N_NODES_K - NG_WIN), 8)
    pltpu.sync_copy(x_hbm.at[pl.ds(nstart, NG_WIN)], nx_v)

    def nchunk(i, carry):
        off = pl.multiple_of(i * NG_CHUNK, 8)
        pltpu.async_copy(nt_hbm.at[nx_v.at[pl.ds(off, NG_CHUNK)]],
                         nrows_v, sem).wait()
        pltpu.sync_copy(nrows_v, hn_hbm.at[pl.ds(nstart + off, NG_CHUNK)])
        return carry
    lax.fori_loop(0, NG_WIN // NG_CHUNK, nchunk, 0)

    ebase = w * E_PER_GTILE

    def echunk(i, carry):
        off = pl.multiple_of(ebase + i * EG_CHUNK, 8)
        pltpu.sync_copy(ea_hbm.at[pl.ds(off, EG_CHUNK)], ex_v)
        pltpu.async_copy(et_hbm.at[ex_v], erows_v, sem).wait()
        pltpu.sync_copy(erows_v, he_hbm.at[pl.ds(off, EG_CHUNK)])
        return carry
    lax.fori_loop(0, E_PER_GTILE // EG_CHUNK, echunk, 0)


_gather = pl.kernel(
    _gather_body,
    out_type=(jax.ShapeDtypeStruct((N_NODES_K, DIM_K), jnp.float32),
              jax.ShapeDtypeStruct((N_EDGES_K, EDIM_K), jnp.float32)),
    mesh=_sc_mesh,
    compiler_params=pltpu.CompilerParams(use_tc_tiling_on_sc=False),
    scratch_types=[
        pltpu.VMEM((NG_WIN,), jnp.int32),
        pltpu.VMEM((NG_CHUNK, DIM_K), jnp.float32),
        pltpu.VMEM((EG_CHUNK,), jnp.int32),
        pltpu.VMEM((EG_CHUNK, EDIM_K), jnp.float32),
        pltpu.SemaphoreType.DMA,
    ],
)


def kernel(x, edge_index, edge_attr, node_table, edge_table,
           node_gamma, node_beta, edge_gamma, edge_beta):
    xi = x.astype(jnp.int32)
    eai = edge_attr.astype(jnp.int32)
    cn, ce = _hist(xi, eai)
    ss = _nstats(cn.reshape(NBLK, 1, VBLK), node_table)
    ntab = _nnorm(ss, node_gamma.reshape(1, DIM_K),
                  node_beta.reshape(1, DIM_K), node_table)
    etab = _enorm(ce.reshape(1, EV_K), edge_gamma.reshape(1, EDIM_K),
                  edge_beta.reshape(1, EDIM_K), edge_table)
    h_node, h_edge = _gather(ntab, xi, etab, eai)
    return (h_node, edge_index, h_edge)


# speedup vs baseline: 9.4559x; 1.1482x over previous
import jax
import jax.numpy as jnp
from jax import lax
from jax.experimental import pallas as pl
from jax.experimental.pallas import tpu as pltpu
from jax.experimental.pallas import tpu_sc as plsc

N_NODES_K = 100000
N_EDGES_K = 3200000
DIM_K = 128
EDIM_K = 16
NV_K = 50000
EV_K = 5000
EPS_K = 1e-5

NC = 2
NS = 16
NW = NC * NS
L = 16


EV_PAD = 5120
NV_PAD = 50176
E_PER_HTILE = N_EDGES_K // NS
E_HCHUNK = 8000
NX_NOM = N_NODES_K // NS
NX_WIN = 6256
NCOPY_W = 3136
ECOPY_W = EV_PAD // NS

_sc_mesh = plsc.VectorSubcoreMesh(core_axis_name="c", subcore_axis_name="s")


def _hist_body(x_hbm, ea_hbm, cn_hbm, ce_hbm,
               ehist_v, exbuf_v, nxbuf_v, nvals_v, zbuf_v, acc_v, tmp_v,
               sh_e, sh_n):
    c = lax.axis_index("c")
    s = lax.axis_index("s")
    is_edge = c == 0

    @pl.when(is_edge)
    def _():
        def zero_body(j, carry):
            ehist_v[pl.ds(j * L, L)] = jnp.zeros((L,), jnp.float32)
            return carry
        lax.fori_loop(0, EV_PAD // L, zero_body, 0)

    @pl.when(jnp.logical_not(is_edge))
    def _():
        def zb(j, carry):
            zbuf_v[pl.ds(j * L, L)] = jnp.zeros((L,), jnp.float32)
            return carry
        lax.fori_loop(0, NCOPY_W // L, zb, 0)
        pltpu.sync_copy(zbuf_v, sh_n.at[pl.ds(s * NCOPY_W, NCOPY_W)])
        lo = s * NX_NOM
        start = pl.multiple_of(lo - lax.rem(lo, 8), 8)
        pltpu.sync_copy(x_hbm.at[pl.ds(start, NX_WIN)], nxbuf_v)
        hi = lo + NX_NOM
        lane = lax.iota(jnp.int32, L)
        def vb(j, carry):
            pos = start + j * L + lane
            m = (pos >= lo) & (pos < hi)
            nvals_v[pl.ds(j * L, L)] = jnp.where(m, 1.0, 0.0).astype(jnp.float32)
            return carry
        lax.fori_loop(0, NX_WIN // L, vb, 0)

    plsc.subcore_barrier()

    @pl.when(is_edge)
    def _():
        base = s * E_PER_HTILE
        ones = jnp.ones((L,), jnp.float32)
        def chunk(i, carry):
            pltpu.sync_copy(ea_hbm.at[pl.ds(base + i * E_HCHUNK, E_HCHUNK)],
                            exbuf_v)
            def inner(j, carry2):
                iv = exbuf_v[pl.ds(j * L, L)]
                plsc.addupdate_scatter(ehist_v, [iv], ones)
                return carry2
            lax.fori_loop(0, E_HCHUNK // L, inner, 0)
            return carry
        lax.fori_loop(0, E_PER_HTILE // E_HCHUNK, chunk, 0)
        pltpu.sync_copy(ehist_v,
                        sh_e.at[pl.ds(pl.multiple_of(s * EV_PAD, 8), EV_PAD)])

    @pl.when(jnp.logical_not(is_edge))
    def _():
        pltpu.sync_copy(nvals_v, sh_n.at[nxbuf_v], add=True)

    plsc.subcore_barrier()

    @pl.when(is_edge)
    def _():
        estart = pl.multiple_of(
            jnp.minimum(s * ECOPY_W, EV_K - ECOPY_W), 8)
        def zb(j, carry):
            acc_v[pl.ds(j * L, L)] = jnp.zeros((L,), jnp.float32)
            return carry
        lax.fori_loop(0, ECOPY_W // L, zb, 0)
        def red(k, carry):
            pltpu.sync_copy(
                sh_e.at[pl.ds(pl.multiple_of(k * EV_PAD + estart, 8),
                              ECOPY_W)], tmp_v)
            def addv(j, carry2):
                acc_v[pl.ds(j * L, L)] = (acc_v[pl.ds(j * L, L)]
                                          + tmp_v[pl.ds(j * L, L)])
                return carry2
            lax.fori_loop(0, ECOPY_W // L, addv, 0)
            return carry
        lax.fori_loop(0, NS, red, 0)
        pltpu.sync_copy(acc_v, ce_hbm.at[pl.ds(estart, ECOPY_W)])

    @pl.when(jnp.logical_not(is_edge))
    def _():
        nlo = s * (NV_K // NS)
        nstart = pl.multiple_of(
            jnp.minimum(nlo - lax.rem(nlo, 8), NV_K - NCOPY_W), 8)
        pltpu.sync_copy(sh_n.at[pl.ds(nstart, NCOPY_W)], zbuf_v)
        pltpu.sync_copy(zbuf_v, cn_hbm.at[pl.ds(nstart, NCOPY_W)])


_hist = pl.kernel(
    _hist_body,
    out_type=(jax.ShapeDtypeStruct((NV_K,), jnp.float32),
              jax.ShapeDtypeStruct((EV_K,), jnp.float32)),
    mesh=_sc_mesh,
    compiler_params=pltpu.CompilerParams(needs_layout_passes=False),
    scratch_types=[
        pltpu.VMEM((EV_PAD,), jnp.float32),
        pltpu.VMEM((E_HCHUNK,), jnp.int32),
        pltpu.VMEM((NX_WIN,), jnp.int32),
        pltpu.VMEM((NX_WIN,), jnp.float32),
        pltpu.VMEM((NCOPY_W,), jnp.float32),
        pltpu.VMEM((ECOPY_W,), jnp.float32),
        pltpu.VMEM((ECOPY_W,), jnp.float32),
        pltpu.VMEM_SHARED((NS * EV_PAD,), jnp.float32),
        pltpu.VMEM_SHARED((NV_PAD,), jnp.float32),
    ],
)


VBLK = 400
NBLK = NV_K // VBLK


def _nstats_body(c_ref, t_ref, o_ref, acc):
    j = pl.program_id(0)

    @pl.when(j == 0)
    def _():
        acc[...] = jnp.zeros_like(acc)

    cvec = c_ref[0]
    T = t_ref[...]
    s1 = jnp.dot(cvec, T, preferred_element_type=jnp.float32)
    s2 = jnp.dot(cvec, T * T, preferred_element_type=jnp.float32)
    acc[0:1, :] = acc[0:1, :] + s1
    acc[1:2, :] = acc[1:2, :] + s2
    o_ref[...] = acc[...]


_nstats = pl.pallas_call(
    _nstats_body,
    grid=(NBLK,),
    in_specs=[pl.BlockSpec((1, 1, VBLK), lambda j: (j, 0, 0)),
              pl.BlockSpec((VBLK, DIM_K), lambda j: (j, 0))],
    out_specs=pl.BlockSpec((2, DIM_K), lambda j: (0, 0)),
    out_shape=jax.ShapeDtypeStruct((2, DIM_K), jnp.float32),
    scratch_shapes=[pltpu.VMEM((2, DIM_K), jnp.float32)],
)


def _nnorm_body(ss_ref, g_ref, b_ref, t_ref, o_ref):
    s1 = ss_ref[0:1, :]
    s2 = ss_ref[1:2, :]
    mean = s1 / N_NODES_K
    var = s2 / N_NODES_K - mean * mean
    a = g_ref[...] * lax.rsqrt(var + EPS_K)
    b = b_ref[...] - mean * a
    o_ref[...] = t_ref[...] * a + b


_nnorm = pl.pallas_call(
    _nnorm_body,
    grid=(NBLK,),
    in_specs=[pl.BlockSpec((2, DIM_K), lambda j: (0, 0)),
              pl.BlockSpec((1, DIM_K), lambda j: (0, 0)),
              pl.BlockSpec((1, DIM_K), lambda j: (0, 0)),
              pl.BlockSpec((VBLK, DIM_K), lambda j: (j, 0))],
    out_specs=pl.BlockSpec((VBLK, DIM_K), lambda j: (j, 0)),
    out_shape=jax.ShapeDtypeStruct((NV_K, DIM_K), jnp.float32),
)


EV_GPAD = 5120


def _enorm_body(c_ref, g_ref, b_ref, t_ref, o_ref):
    cvec = c_ref[...]
    Tt = t_ref[...]
    s1 = jnp.dot(Tt, cvec, preferred_element_type=jnp.float32)
    s2 = jnp.dot(Tt * Tt, cvec, preferred_element_type=jnp.float32)
    mean = s1 / N_EDGES_K
    var = s2 / N_EDGES_K - mean * mean
    a = g_ref[...] * lax.rsqrt(var + EPS_K)
    b = b_ref[...] - mean * a
    o_ref[...] = Tt * a + b


_enorm = pl.pallas_call(
    _enorm_body,
    out_shape=jax.ShapeDtypeStruct((EDIM_K, EV_GPAD), jnp.float32),
)


NG_NOM = N_NODES_K // NW
NG_WIN = 3136
NG_CHUNK = 392
NGRP = N_EDGES_K // 128
G_HALF = NGRP // 2
EG_GRP = 20
EG_CHUNK = EG_GRP * 128
EG_STEPS = G_HALF // EG_GRP


def _gather_body(nt_hbm, x_hbm, et_hbm, ea_hbm, hn_hbm, he4_hbm,
                 nx_v, nrows_v, ex_v, trow_v, obuf_v, sem):
    c = lax.axis_index("c")
    s = lax.axis_index("s")
    w = s * NC + c

    nlo = w * NG_NOM
    nstart = pl.multiple_of(
        jnp.minimum(nlo - lax.rem(nlo, 8), N_NODES_K - NG_WIN), 8)
    pltpu.sync_copy(x_hbm.at[pl.ds(nstart, NG_WIN)], nx_v)

    def nchunk(i, carry):
        off = pl.multiple_of(i * NG_CHUNK, 8)
        pltpu.async_copy(nt_hbm.at[nx_v.at[pl.ds(off, NG_CHUNK)]],
                         nrows_v, sem).wait()
        pltpu.sync_copy(nrows_v, hn_hbm.at[pl.ds(nstart + off, NG_CHUNK)])
        return carry
    lax.fori_loop(0, NG_WIN // NG_CHUNK, nchunk, 0)

    f = s
    band = s // 8
    fr = s - band * 8
    pltpu.sync_copy(et_hbm.at[f], trow_v)

    def echunk(i, carry):
        g0 = c * G_HALF + i * EG_GRP
        eoff = pl.multiple_of(g0 * 128, 8)
        pltpu.sync_copy(ea_hbm.at[pl.ds(eoff, EG_CHUNK)], ex_v)

        def inner(j, carry2):
            iv = ex_v[pl.ds(j * L, L)]
            vals = plsc.load_gather(trow_v, [iv])
            jr = j // 8
            jc = (j - jr * 8) * L
            obuf_v[jr, pl.ds(jc, L)] = vals
            return carry2
        lax.fori_loop(0, EG_CHUNK // L, inner, 0)
        pltpu.sync_copy(obuf_v, he4_hbm.at[band, pl.ds(g0, EG_GRP), fr])
        return carry
    lax.fori_loop(0, EG_STEPS, echunk, 0)


_gather = pl.kernel(
    _gather_body,
    out_type=(jax.ShapeDtypeStruct((N_NODES_K, DIM_K), jnp.float32),
              jax.ShapeDtypeStruct((2, NGRP, 8, 128), jnp.float32)),
    mesh=_sc_mesh,
    compiler_params=pltpu.CompilerParams(use_tc_tiling_on_sc=False,
                                         needs_layout_passes=False),
    scratch_types=[
        pltpu.VMEM((NG_WIN,), jnp.int32),
        pltpu.VMEM((NG_CHUNK, DIM_K), jnp.float32),
        pltpu.VMEM((EG_CHUNK,), jnp.int32),
        pltpu.VMEM((EV_GPAD,), jnp.float32),
        pltpu.VMEM((EG_GRP, 128), jnp.float32),
        pltpu.SemaphoreType.DMA,
    ],
)


def kernel(x, edge_index, edge_attr, node_table, edge_table,
           node_gamma, node_beta, edge_gamma, edge_beta):
    xi = x.astype(jnp.int32)
    eai = edge_attr.astype(jnp.int32)
    cn, ce = _hist(xi, eai)
    ss = _nstats(cn.reshape(NBLK, 1, VBLK), node_table)
    ntab = _nnorm(ss, node_gamma.reshape(1, DIM_K),
                  node_beta.reshape(1, DIM_K), node_table)
    ce_pad = jnp.pad(ce, (0, EV_GPAD - EV_K)).reshape(EV_GPAD, 1)
    et_t = jnp.pad(edge_table.T, ((0, 0), (0, EV_GPAD - EV_K)))
    etab_t = _enorm(ce_pad, edge_gamma.reshape(EDIM_K, 1),
                    edge_beta.reshape(EDIM_K, 1), et_t)
    h_node, he4 = _gather(ntab, xi, etab_t, eai)
    h_edge = he4.transpose(1, 3, 0, 2).reshape(N_EDGES_K, EDIM_K)
    return (h_node, edge_index, h_edge)
